# baseline (speedup 1.0000x reference)
import functools

import jax
import jax.numpy as jnp
from jax import lax
from jax.experimental import pallas as pl
from jax.experimental.pallas import tpu as pltpu
from jax.experimental.pallas import tpu_sc as plsc

SEQ_LEN = 200
BATCH = 4096
EMB_DIM = 64
N_CLASSES = 16
VOCAB = 1_000_000

TOK_BLK = 32_768
PROJ_GRID = -(-VOCAB // TOK_BLK)
VOCAB_PAD = PROJ_GRID * TOK_BLK


def _proj_body(tt_ref, w_ref, d_ref, out_ref, d2_ref, pt_ref):
    pt_ref[...] = jax.lax.dot_general(
        w_ref[...], tt_ref[...],
        dimension_numbers=(((1,), (0,)), ((), ())),
        preferred_element_type=jnp.float32,
    )
    for j in range(TOK_BLK // 1024):
        m = jnp.concatenate(
            [pt_ref[:, j * 1024 + a * 128:j * 1024 + (a + 1) * 128]
             for a in range(8)], axis=0)
        out_ref[j * 128:(j + 1) * 128, :] = m.T

    @pl.when(pl.program_id(0) == PROJ_GRID - 1)
    def _():
        d = d_ref[...]
        d2_ref[...] = (d & ~1023) | ((d & 127) << 3) | ((d >> 7) & 7)


_proj = pl.pallas_call(
    _proj_body,
    grid=(PROJ_GRID,),
    in_specs=[
        pl.BlockSpec((EMB_DIM, TOK_BLK), lambda i: (0, i)),
        pl.BlockSpec((N_CLASSES, EMB_DIM), lambda i: (0, 0)),
        pl.BlockSpec((SEQ_LEN, BATCH), lambda i: (0, 0)),
    ],
    out_specs=[
        pl.BlockSpec((TOK_BLK // 8, 128), lambda i: (i, 0)),
        pl.BlockSpec((SEQ_LEN, BATCH), lambda i: (0, 0)),
    ],
    out_shape=[
        jax.ShapeDtypeStruct((VOCAB_PAD // 8, 128), jnp.float32),
        jax.ShapeDtypeStruct((SEQ_LEN, BATCH), jnp.int32),
    ],
    scratch_shapes=[pltpu.VMEM((N_CLASSES, TOK_BLK), jnp.float32)],
    compiler_params=pltpu.CompilerParams(fuse_transposed_lhs_in_matmul=True),
)

NUM_CORES = 2
NUM_SUBCORES = 16
NUM_WORKERS = NUM_CORES * NUM_SUBCORES
BPW = BATCH // NUM_WORKERS

NBUF = 20
NDMA = SEQ_LEN

_mesh = plsc.VectorSubcoreMesh(
    core_axis_name="c", subcore_axis_name="s",
    num_cores=NUM_CORES, num_subcores=NUM_SUBCORES,
)


@functools.partial(
    pl.kernel,
    out_type=jax.ShapeDtypeStruct((BATCH, N_CLASSES), jnp.float32),
    mesh=_mesh,
    compiler_params=pltpu.CompilerParams(use_tc_tiling_on_sc=False),
    scratch_types=[
        pltpu.VMEM((NDMA, BPW), jnp.int32),
        [pltpu.VMEM((BPW, N_CLASSES), jnp.float32) for _ in range(NBUF)],
        pltpu.VMEM((BPW, N_CLASSES), jnp.float32),
        pltpu.VMEM((N_CLASSES,), jnp.float32),
        [pltpu.SemaphoreType.DMA for _ in range(NBUF)],
    ],
)
def _pool(p_hbm, docs_hbm, b_hbm, out_hbm, idx_v, accs, out_v, b_v, sems):
    wid = lax.axis_index("s") * NUM_CORES + lax.axis_index("c")
    base = wid * BPW

    pltpu.sync_copy(docs_hbm.at[:, pl.ds(base, BPW)], idx_v)
    pltpu.sync_copy(b_hbm, b_v)

    def gather(j, k, add):
        idx = idx_v.at[j]
        pltpu.async_copy(p_hbm.at[idx], accs[k], sems[k], add=add)

    def wait(j, k):
        idx = idx_v.at[j]
        pltpu.make_async_copy(p_hbm.at[idx], accs[k], sems[k]).wait()

    for k in range(NBUF):
        gather(k, k, False)

    @pl.loop(1, NDMA // NBUF)
    def _(g):
        for k in range(NBUF):
            wait(NBUF * (g - 1) + k, k)
            gather(NBUF * g + k, k, True)

    for k in range(NBUF):
        wait(NDMA - NBUF + k, k)

    inv_len = 1.0 / SEQ_LEN
    bias = b_v[...]

    @pl.loop(0, BPW)
    def _(i):
        s = accs[0][i, :]
        for k in range(1, NBUF):
            s = s + accs[k][i, :]
        out_v[i, :] = s * inv_len + bias

    pltpu.sync_copy(out_v, out_hbm.at[pl.ds(base, BPW)])


@jax.jit
def kernel(docs, emb_table, W, b):
    p2, docs2 = _proj(emb_table.T, W, docs)
    p = p2.reshape(VOCAB_PAD, N_CLASSES)
    return _pool(p, docs2, b)


# speedup vs baseline: 1.0031x; 1.0031x over previous
import functools

import jax
import jax.numpy as jnp
from jax import lax
from jax.experimental import pallas as pl
from jax.experimental.pallas import tpu as pltpu
from jax.experimental.pallas import tpu_sc as plsc

SEQ_LEN = 200
BATCH = 4096
EMB_DIM = 64
N_CLASSES = 16
VOCAB = 1_000_000

TOK_BLK = 32_768
PROJ_GRID = -(-VOCAB // TOK_BLK)
VOCAB_PAD = PROJ_GRID * TOK_BLK


def _proj_body(tt_ref, w_ref, d_ref, out_ref, d2_ref, pt_ref):
    pt_ref[...] = jax.lax.dot_general(
        w_ref[...], tt_ref[...],
        dimension_numbers=(((1,), (0,)), ((), ())),
        preferred_element_type=jnp.float32,
    )
    for j in range(TOK_BLK // 1024):
        m = jnp.concatenate(
            [pt_ref[:, j * 1024 + a * 128:j * 1024 + (a + 1) * 128]
             for a in range(8)], axis=0)
        out_ref[j * 128:(j + 1) * 128, :] = m.T

    @pl.when(pl.program_id(0) == PROJ_GRID - 1)
    def _():
        d = d_ref[...]
        d2_ref[...] = (d & ~1023) | ((d & 127) << 3) | ((d >> 7) & 7)


_proj = pl.pallas_call(
    _proj_body,
    grid=(PROJ_GRID,),
    in_specs=[
        pl.BlockSpec((EMB_DIM, TOK_BLK), lambda i: (0, i)),
        pl.BlockSpec((N_CLASSES, EMB_DIM), lambda i: (0, 0)),
        pl.BlockSpec((SEQ_LEN, BATCH), lambda i: (0, 0)),
    ],
    out_specs=[
        pl.BlockSpec((TOK_BLK // 8, 128), lambda i: (i, 0)),
        pl.BlockSpec((SEQ_LEN, BATCH), lambda i: (0, 0)),
    ],
    out_shape=[
        jax.ShapeDtypeStruct((VOCAB_PAD // 8, 128), jnp.float32),
        jax.ShapeDtypeStruct((SEQ_LEN, BATCH), jnp.int32),
    ],
    scratch_shapes=[pltpu.VMEM((N_CLASSES, TOK_BLK), jnp.float32)],
    compiler_params=pltpu.CompilerParams(fuse_transposed_lhs_in_matmul=True),
)

NUM_CORES = 2
NUM_SUBCORES = 16
NUM_WORKERS = NUM_CORES * NUM_SUBCORES
BPW = BATCH // NUM_WORKERS

NBUF = 10
NDMA = SEQ_LEN

_mesh = plsc.VectorSubcoreMesh(
    core_axis_name="c", subcore_axis_name="s",
    num_cores=NUM_CORES, num_subcores=NUM_SUBCORES,
)


@functools.partial(
    pl.kernel,
    out_type=jax.ShapeDtypeStruct((BATCH, N_CLASSES), jnp.float32),
    mesh=_mesh,
    compiler_params=pltpu.CompilerParams(use_tc_tiling_on_sc=False),
    scratch_types=[
        pltpu.VMEM((NDMA, BPW), jnp.int32),
        [pltpu.VMEM((BPW, N_CLASSES), jnp.float32) for _ in range(NBUF)],
        pltpu.VMEM((BPW, N_CLASSES), jnp.float32),
        pltpu.VMEM((N_CLASSES,), jnp.float32),
        [pltpu.SemaphoreType.DMA for _ in range(NBUF)],
    ],
)
def _pool(p_hbm, docs_hbm, b_hbm, out_hbm, idx_v, accs, out_v, b_v, sems):
    wid = lax.axis_index("s") * NUM_CORES + lax.axis_index("c")
    base = wid * BPW

    pltpu.sync_copy(docs_hbm.at[:, pl.ds(base, BPW)], idx_v)
    pltpu.sync_copy(b_hbm, b_v)

    def gather(j, k, add):
        idx = idx_v.at[j]
        pltpu.async_copy(p_hbm.at[idx], accs[k], sems[k], add=add)

    def wait(j, k):
        idx = idx_v.at[j]
        pltpu.make_async_copy(p_hbm.at[idx], accs[k], sems[k]).wait()

    for k in range(NBUF):
        gather(k, k, False)

    @pl.loop(1, NDMA // NBUF)
    def _(g):
        for k in range(NBUF):
            wait(NBUF * (g - 1) + k, k)
            gather(NBUF * g + k, k, True)

    for k in range(NBUF):
        wait(NDMA - NBUF + k, k)

    inv_len = 1.0 / SEQ_LEN
    bias = b_v[...]

    @pl.loop(0, BPW)
    def _(i):
        s = accs[0][i, :]
        for k in range(1, NBUF):
            s = s + accs[k][i, :]
        out_v[i, :] = s * inv_len + bias

    pltpu.sync_copy(out_v, out_hbm.at[pl.ds(base, BPW)])


@jax.jit
def kernel(docs, emb_table, W, b):
    p2, docs2 = _proj(emb_table.T, W, docs)
    p = p2.reshape(VOCAB_PAD, N_CLASSES)
    return _pool(p, docs2, b)


# speedup vs baseline: 1.0087x; 1.0056x over previous
import functools

import jax
import jax.numpy as jnp
from jax import lax
from jax.experimental import pallas as pl
from jax.experimental.pallas import tpu as pltpu
from jax.experimental.pallas import tpu_sc as plsc

SEQ_LEN = 200
BATCH = 4096
EMB_DIM = 64
N_CLASSES = 16
VOCAB = 1_000_000

TOK_BLK = 32_768
PROJ_GRID = -(-VOCAB // TOK_BLK)
VOCAB_PAD = PROJ_GRID * TOK_BLK


def _proj_body(tt_ref, w_ref, d_ref, out_ref, d2_ref, pt_ref):
    pt_ref[...] = jax.lax.dot_general(
        w_ref[...], tt_ref[...],
        dimension_numbers=(((1,), (0,)), ((), ())),
        preferred_element_type=jnp.float32,
    )
    for j in range(TOK_BLK // 1024):
        m = jnp.concatenate(
            [pt_ref[:, j * 1024 + a * 128:j * 1024 + (a + 1) * 128]
             for a in range(8)], axis=0)
        out_ref[j * 128:(j + 1) * 128, :] = m.T

    @pl.when(pl.program_id(0) == PROJ_GRID - 1)
    def _():
        d = d_ref[...]
        d2_ref[...] = (d & ~1023) | ((d & 127) << 3) | ((d >> 7) & 7)


_proj = pl.pallas_call(
    _proj_body,
    grid=(PROJ_GRID,),
    in_specs=[
        pl.BlockSpec((EMB_DIM, TOK_BLK), lambda i: (0, i)),
        pl.BlockSpec((N_CLASSES, EMB_DIM), lambda i: (0, 0)),
        pl.BlockSpec((SEQ_LEN, BATCH), lambda i: (0, 0)),
    ],
    out_specs=[
        pl.BlockSpec((TOK_BLK // 8, 128), lambda i: (i, 0)),
        pl.BlockSpec((SEQ_LEN, BATCH), lambda i: (0, 0)),
    ],
    out_shape=[
        jax.ShapeDtypeStruct((VOCAB_PAD // 8, 128), jnp.float32),
        jax.ShapeDtypeStruct((SEQ_LEN, BATCH), jnp.int32),
    ],
    scratch_shapes=[pltpu.VMEM((N_CLASSES, TOK_BLK), jnp.float32)],
    compiler_params=pltpu.CompilerParams(fuse_transposed_lhs_in_matmul=True),
)

NUM_CORES = 2
NUM_SUBCORES = 16
NUM_WORKERS = NUM_CORES * NUM_SUBCORES
BPW = BATCH // NUM_WORKERS

NBUF = 8
NDMA = SEQ_LEN

_mesh = plsc.VectorSubcoreMesh(
    core_axis_name="c", subcore_axis_name="s",
    num_cores=NUM_CORES, num_subcores=NUM_SUBCORES,
)


@functools.partial(
    pl.kernel,
    out_type=jax.ShapeDtypeStruct((BATCH, N_CLASSES), jnp.float32),
    mesh=_mesh,
    compiler_params=pltpu.CompilerParams(use_tc_tiling_on_sc=False),
    scratch_types=[
        pltpu.VMEM((NDMA, BPW), jnp.int32),
        [pltpu.VMEM((BPW, N_CLASSES), jnp.float32) for _ in range(NBUF)],
        pltpu.VMEM((BPW, N_CLASSES), jnp.float32),
        pltpu.VMEM((N_CLASSES,), jnp.float32),
        [pltpu.SemaphoreType.DMA for _ in range(NBUF)],
    ],
)
def _pool(p_hbm, docs_hbm, b_hbm, out_hbm, idx_v, accs, out_v, b_v, sems):
    wid = lax.axis_index("s") * NUM_CORES + lax.axis_index("c")
    base = wid * BPW

    pltpu.sync_copy(docs_hbm.at[:, pl.ds(base, BPW)], idx_v)
    pltpu.sync_copy(b_hbm, b_v)

    def gather(j, k, add):
        idx = idx_v.at[j]
        pltpu.async_copy(p_hbm.at[idx], accs[k], sems[k], add=add)

    def wait(j, k):
        idx = idx_v.at[j]
        pltpu.make_async_copy(p_hbm.at[idx], accs[k], sems[k]).wait()

    for k in range(NBUF):
        gather(k, k, False)

    @pl.loop(1, NDMA // NBUF)
    def _(g):
        for k in range(NBUF):
            wait(NBUF * (g - 1) + k, k)
            gather(NBUF * g + k, k, True)

    for k in range(NBUF):
        wait(NDMA - NBUF + k, k)

    inv_len = 1.0 / SEQ_LEN
    bias = b_v[...]

    @pl.loop(0, BPW)
    def _(i):
        s = accs[0][i, :]
        for k in range(1, NBUF):
            s = s + accs[k][i, :]
        out_v[i, :] = s * inv_len + bias

    pltpu.sync_copy(out_v, out_hbm.at[pl.ds(base, BPW)])


@jax.jit
def kernel(docs, emb_table, W, b):
    p2, docs2 = _proj(emb_table.T, W, docs)
    p = p2.reshape(VOCAB_PAD, N_CLASSES)
    return _pool(p, docs2, b)
